# baseline (speedup 1.0000x reference)
import functools

import jax
import jax.numpy as jnp
from jax import lax
from jax.experimental import pallas as pl
from jax.experimental.pallas import tpu as pltpu
from jax.experimental.pallas import tpu_sc as plsc

_ALPHA = 0.1
_BETA = 1.0
_L1 = 1.0
_L2 = 1.0
_D = 1000000
_B = 16384
_F_W = 100

_NC, _NS = 2, 16
_NW = _NC * _NS
_ROWS = _B // _NW
_CH = (4, 8, 8, 20, 20, 20, 20)
_NCHUNK = len(_CH)
_COFF = tuple(sum(_CH[:i]) for i in range(_NCHUNK))
_FMAX = max(_CH)
_CELEMS = _FMAX * _ROWS
_SH = 62504
_BNC = 10240


def _wtab_body(z_ref, n_ref, o_ref):
    zv = z_ref[...]
    nv = n_ref[...]
    sign = jnp.where(zv < 0, -1.0, 1.0)
    denom = (_BETA + jnp.sqrt(nv)) / _ALPHA + _L2
    o_ref[...] = jnp.where(sign * zv <= _L1, 0.0, (sign * _L1 - zv) / denom)


_WBLK = 131072

_wtab = pl.pallas_call(
    _wtab_body,
    grid=(8,),
    in_specs=[
        pl.BlockSpec((_WBLK,), lambda i: (i,)),
        pl.BlockSpec((_WBLK,), lambda i: (i,)),
    ],
    out_specs=pl.BlockSpec((_WBLK,), lambda i: (i,)),
    out_shape=jax.ShapeDtypeStruct((_D,), jnp.float32),
)


@functools.cache
def _build_gather_sc():
    mesh = plsc.VectorSubcoreMesh(core_axis_name="c", subcore_axis_name="s")

    @functools.partial(
        pl.kernel,
        out_type=jax.ShapeDtypeStruct((_B,), jnp.float32),
        mesh=mesh,
        scratch_types=[
            pltpu.VMEM((_CELEMS,), jnp.int32),
            pltpu.VMEM((_CELEMS,), jnp.int32),
            pltpu.VMEM((_CELEMS,), jnp.int32),
            pltpu.VMEM((_CELEMS,), jnp.float32),
            pltpu.VMEM((_CELEMS,), jnp.float32),
            pltpu.VMEM((_ROWS,), jnp.float32),
            pltpu.VMEM((16,), jnp.float32),
            pltpu.VMEM_SHARED((_D,), jnp.float32),
            pltpu.SemaphoreType.DMA,
            pltpu.SemaphoreType.DMA,
            pltpu.SemaphoreType.DMA,
            pltpu.SemaphoreType.DMA,
            pltpu.SemaphoreType.DMA,
            pltpu.SemaphoreType.DMA,
        ],
    )
    def _gather_sc(idx_hbm, tab_hbm, b_hbm, out_hbm, idx_a, idx_b, idx_c,
                   val_a, val_b, acc_v, b_v, spm,
                   sem_ia, sem_ib, sem_ic, sem_a, sem_b, sem_s):
        wid = lax.axis_index("s") * _NC + lax.axis_index("c")
        sid = lax.axis_index("s")

        ibufs = (idx_a, idx_b, idx_c)
        isems = (sem_ia, sem_ib, sem_ic)
        vbufs = (val_a, val_b)
        vsems = (sem_a, sem_b)

        def _fire_idx(c):
            buf, sem = ibufs[c % 3], isems[c % 3]
            return [
                pltpu.async_copy(
                    idx_hbm.at[_COFF[c] + j, pl.ds(wid * _ROWS, _ROWS)],
                    buf.at[pl.ds(j * _ROWS, _ROWS)], sem)
                for j in range(_CH[c])
            ]

        soff = sid * _SH

        def _stage(shard):
            nchunks = (shard + _BNC - 1) // _BNC
            szs = [min(_BNC, shard - k * _BNC) for k in range(nchunks)]
            hs = [None] * nchunks
            ws = [None] * nchunks
            hs[0] = pltpu.async_copy(tab_hbm.at[pl.ds(soff, szs[0])],
                                     val_a.at[pl.ds(0, szs[0])], sem_a)
            for k in range(nchunks):
                buf = (val_a, val_b)[k % 2]
                if k >= 1:
                    ws[k - 1].wait()
                if k + 1 < nchunks:
                    nbuf = (val_a, val_b)[(k + 1) % 2]
                    hs[k + 1] = pltpu.async_copy(
                        tab_hbm.at[pl.ds(soff + (k + 1) * _BNC, szs[k + 1])],
                        nbuf.at[pl.ds(0, szs[k + 1])],
                        (sem_a, sem_b)[(k + 1) % 2])
                hs[k].wait()
                ws[k] = pltpu.async_copy(
                    buf.at[pl.ds(0, szs[k])],
                    spm.at[pl.ds(soff + k * _BNC, szs[k])], sem_s)
            ws[nchunks - 1].wait()

        idx_pending = [_fire_idx(0), _fire_idx(1), _fire_idx(2)]

        with jax.named_scope("stage_tab"):
            @pl.when(sid < _NS - 1)
            def _stage_full():
                _stage(_SH)

            @pl.when(sid == _NS - 1)
            def _stage_last():
                _stage(_D - (_NS - 1) * _SH)

        with jax.named_scope("stage_barrier"):
            plsc.subcore_barrier()

        def _fire_g(c):
            buf, sem = vbufs[c % 2], vsems[c % 2]
            ibuf = ibufs[c % 3]
            ne = _CH[c] * _ROWS
            return [
                pltpu.async_copy(spm.at[ibuf.at[pl.ds(0, ne)]],
                                 buf.at[pl.ds(0, ne)], sem)
            ]

        def _reduce(c):
            buf = vbufs[c % 2]

            def _group(g, _):
                off = g * 16
                a = (jnp.zeros((16,), jnp.float32) if c == 0
                     else acc_v[pl.ds(off, 16)])
                for j in range(_CH[c]):
                    a = a + buf[pl.ds(j * _ROWS + off, 16)]
                acc_v[pl.ds(off, 16)] = a
                return 0

            lax.fori_loop(0, _ROWS // 16, _group, 0)

        g_pending = [None] * _NCHUNK
        for c in range(_NCHUNK):
            with jax.named_scope(f"idxwait{c}"):
                for h in idx_pending[c]:
                    h.wait()
            with jax.named_scope(f"fire{c}"):
                g_pending[c] = _fire_g(c)
            if c >= 1:
                with jax.named_scope(f"gwait{c - 1}"):
                    for h in g_pending[c - 1]:
                        h.wait()
                if c + 2 < _NCHUNK:
                    idx_pending.append(_fire_idx(c + 2))
                with jax.named_scope(f"reduce{c - 1}"):
                    _reduce(c - 1)
        with jax.named_scope("gwait_last"):
            for h in g_pending[_NCHUNK - 1]:
                h.wait()
        with jax.named_scope("reduce_last"):
            _reduce(_NCHUNK - 1)

        pltpu.sync_copy(b_hbm, b_v)
        bvec = b_v[...]
        for g in range(_ROWS // 16):
            tot = acc_v[pl.ds(g * 16, 16)]
            x = jnp.maximum(jnp.minimum(tot, 35.0), -35.0) + bvec
            acc_v[pl.ds(g * 16, 16)] = 1.0 / (1.0 + jnp.exp(-x))

        pltpu.sync_copy(acc_v, out_hbm.at[pl.ds(wid * _ROWS, _ROWS)])

    return _gather_sc


def kernel(X_w_indices, X_d, y, z, n, W, b):
    wtab = _wtab(z, n)
    idx_t = X_w_indices.T
    bb = jnp.broadcast_to(b, (16,))
    y_pred = _build_gather_sc()(idx_t, wtab, bb)
    return y_pred.reshape(_B, 1)


# speedup vs baseline: 1.0276x; 1.0276x over previous
import functools

import jax
import jax.numpy as jnp
from jax import lax
from jax.experimental import pallas as pl
from jax.experimental.pallas import tpu as pltpu
from jax.experimental.pallas import tpu_sc as plsc

_ALPHA = 0.1
_BETA = 1.0
_L1 = 1.0
_L2 = 1.0
_D = 1000000
_B = 16384
_F_W = 100

_NC, _NS = 2, 16
_NW = _NC * _NS
_ROWS = _B // _NW
_CH = (4, 8, 8, 20, 20, 20, 20)
_NCHUNK = len(_CH)
_COFF = tuple(sum(_CH[:i]) for i in range(_NCHUNK))
_FMAX = max(_CH)
_CELEMS = _FMAX * _ROWS
_SH = 62504
_BNC = 10240


def _wtab_body(z_ref, n_ref, o_ref):
    zv = z_ref[...]
    nv = n_ref[...]
    sign = jnp.where(zv < 0, -1.0, 1.0)
    denom = (_BETA + jnp.sqrt(nv)) / _ALPHA + _L2
    o_ref[...] = jnp.where(sign * zv <= _L1, 0.0, (sign * _L1 - zv) / denom)


_wtab = pl.pallas_call(
    _wtab_body,
    out_shape=jax.ShapeDtypeStruct((_D,), jnp.float32),
)


@functools.cache
def _build_gather_sc():
    mesh = plsc.VectorSubcoreMesh(core_axis_name="c", subcore_axis_name="s")

    @functools.partial(
        pl.kernel,
        out_type=jax.ShapeDtypeStruct((_B,), jnp.float32),
        mesh=mesh,
        scratch_types=[
            pltpu.VMEM((_CELEMS,), jnp.int32),
            pltpu.VMEM((_CELEMS,), jnp.int32),
            pltpu.VMEM((_CELEMS,), jnp.int32),
            pltpu.VMEM((_CELEMS,), jnp.float32),
            pltpu.VMEM((_CELEMS,), jnp.float32),
            pltpu.VMEM((_ROWS,), jnp.float32),
            pltpu.VMEM((16,), jnp.float32),
            pltpu.VMEM_SHARED((_D,), jnp.float32),
            pltpu.SemaphoreType.DMA,
            pltpu.SemaphoreType.DMA,
            pltpu.SemaphoreType.DMA,
            pltpu.SemaphoreType.DMA,
            pltpu.SemaphoreType.DMA,
            pltpu.SemaphoreType.DMA,
        ],
    )
    def _gather_sc(idx_hbm, tab_hbm, b_hbm, out_hbm, idx_a, idx_b, idx_c,
                   val_a, val_b, acc_v, b_v, spm,
                   sem_ia, sem_ib, sem_ic, sem_a, sem_b, sem_s):
        wid = lax.axis_index("s") * _NC + lax.axis_index("c")
        sid = lax.axis_index("s")

        ibufs = (idx_a, idx_b, idx_c)
        isems = (sem_ia, sem_ib, sem_ic)
        vbufs = (val_a, val_b)
        vsems = (sem_a, sem_b)

        def _fire_idx(c):
            buf, sem = ibufs[c % 3], isems[c % 3]
            return [
                pltpu.async_copy(
                    idx_hbm.at[_COFF[c] + j, pl.ds(wid * _ROWS, _ROWS)],
                    buf.at[pl.ds(j * _ROWS, _ROWS)], sem)
                for j in range(_CH[c])
            ]

        soff = sid * _SH

        def _stage(shard):
            nchunks = (shard + _BNC - 1) // _BNC
            szs = [min(_BNC, shard - k * _BNC) for k in range(nchunks)]
            hs = [None] * nchunks
            ws = [None] * nchunks
            hs[0] = pltpu.async_copy(tab_hbm.at[pl.ds(soff, szs[0])],
                                     val_a.at[pl.ds(0, szs[0])], sem_a)
            for k in range(nchunks):
                buf = (val_a, val_b)[k % 2]
                if k >= 1:
                    ws[k - 1].wait()
                if k + 1 < nchunks:
                    nbuf = (val_a, val_b)[(k + 1) % 2]
                    hs[k + 1] = pltpu.async_copy(
                        tab_hbm.at[pl.ds(soff + (k + 1) * _BNC, szs[k + 1])],
                        nbuf.at[pl.ds(0, szs[k + 1])],
                        (sem_a, sem_b)[(k + 1) % 2])
                hs[k].wait()
                ws[k] = pltpu.async_copy(
                    buf.at[pl.ds(0, szs[k])],
                    spm.at[pl.ds(soff + k * _BNC, szs[k])], sem_s)
            ws[nchunks - 1].wait()

        idx_pending = [_fire_idx(0), _fire_idx(1), _fire_idx(2)]

        with jax.named_scope("stage_tab"):
            @pl.when(sid < _NS - 1)
            def _stage_full():
                _stage(_SH)

            @pl.when(sid == _NS - 1)
            def _stage_last():
                _stage(_D - (_NS - 1) * _SH)

        with jax.named_scope("stage_barrier"):
            plsc.subcore_barrier()

        def _fire_g(c):
            buf, sem = vbufs[c % 2], vsems[c % 2]
            ibuf = ibufs[c % 3]
            ne = _CH[c] * _ROWS
            return [
                pltpu.async_copy(spm.at[ibuf.at[pl.ds(0, ne)]],
                                 buf.at[pl.ds(0, ne)], sem)
            ]

        def _reduce(c):
            buf = vbufs[c % 2]

            def _group(g, _):
                off = g * 16
                a = (jnp.zeros((16,), jnp.float32) if c == 0
                     else acc_v[pl.ds(off, 16)])
                for j in range(_CH[c]):
                    a = a + buf[pl.ds(j * _ROWS + off, 16)]
                acc_v[pl.ds(off, 16)] = a
                return 0

            lax.fori_loop(0, _ROWS // 16, _group, 0)

        g_pending = [None] * _NCHUNK
        for c in range(_NCHUNK):
            with jax.named_scope(f"idxwait{c}"):
                for h in idx_pending[c]:
                    h.wait()
            with jax.named_scope(f"fire{c}"):
                g_pending[c] = _fire_g(c)
            if c >= 1:
                with jax.named_scope(f"gwait{c - 1}"):
                    for h in g_pending[c - 1]:
                        h.wait()
                if c + 2 < _NCHUNK:
                    idx_pending.append(_fire_idx(c + 2))
                with jax.named_scope(f"reduce{c - 1}"):
                    _reduce(c - 1)
        with jax.named_scope("gwait_last"):
            for h in g_pending[_NCHUNK - 1]:
                h.wait()
        with jax.named_scope("reduce_last"):
            _reduce(_NCHUNK - 1)

        pltpu.sync_copy(b_hbm, b_v)
        bvec = b_v[...]
        for g in range(_ROWS // 16):
            tot = acc_v[pl.ds(g * 16, 16)]
            x = jnp.maximum(jnp.minimum(tot, 35.0), -35.0) + bvec
            acc_v[pl.ds(g * 16, 16)] = 1.0 / (1.0 + jnp.exp(-x))

        pltpu.sync_copy(acc_v, out_hbm.at[pl.ds(wid * _ROWS, _ROWS)])

    return _gather_sc


def kernel(X_w_indices, X_d, y, z, n, W, b):
    wtab = _wtab(z, n)
    idx_t = X_w_indices.T
    bb = jnp.broadcast_to(b, (16,))
    y_pred = _build_gather_sc()(idx_t, wtab, bb)
    return y_pred.reshape(_B, 1)
